# baseline (speedup 1.0000x reference)
import functools

import jax
import jax.numpy as jnp
from jax.experimental import pallas as pl
from jax.experimental.pallas import tpu as pltpu

TILE = 256
FB = 512


def _router_kernel(x_ref, g_ref, idx_ref, w_ref, *, n_exp):
    logits = jnp.dot(x_ref[...], g_ref[...], preferred_element_type=jnp.float32)
    eidx = jax.lax.broadcasted_iota(jnp.int32, logits.shape, 1)
    m1 = jnp.max(logits, axis=1, keepdims=True)
    i1 = jnp.min(jnp.where(logits == m1, eidx, n_exp), axis=1, keepdims=True)
    masked = jnp.where(eidx == i1, -jnp.inf, logits)
    m2 = jnp.max(masked, axis=1, keepdims=True)
    i2 = jnp.min(jnp.where(masked == m2, eidx, n_exp), axis=1, keepdims=True)
    e2 = jnp.exp(m2 - m1)
    w1 = 1.0 / (1.0 + e2)
    w2 = e2 / (1.0 + e2)
    idx_ref[...] = jnp.concatenate([i1, i2], axis=1)
    w_ref[...] = jnp.concatenate([w1, w2], axis=1)


def _gmm_kernel(grp_ref, tok_ref,
                x_ref, wi0_ref, wi1_ref, wo_ref, w_ref,
                out_ref,
                xs_scr, acc_scr, row_scr, *, nf):
    i = pl.program_id(0)
    f = pl.program_id(1)

    @pl.when(jnp.logical_and(i == 0, f == 0))
    def _():
        out_ref[...] = jnp.zeros_like(out_ref)

    @pl.when(f == 0)
    def _():
        def gather_body(r, _):
            tok = tok_ref[i * TILE + r]
            xs_scr[r, :] = x_ref[tok, :]
            return 0
        jax.lax.fori_loop(0, TILE, gather_body, 0, unroll=8)

    xs = xs_scr[...].astype(jnp.bfloat16)
    a0 = jnp.dot(xs, wi0_ref[0], preferred_element_type=jnp.float32)
    a1 = jnp.dot(xs, wi1_ref[0], preferred_element_type=jnp.float32)
    h = (jax.nn.gelu(a0) * a1).astype(jnp.bfloat16)
    contrib = jnp.dot(h, wo_ref[0], preferred_element_type=jnp.float32)

    @pl.when(f == 0)
    def _():
        acc_scr[...] = contrib

    @pl.when(f != 0)
    def _():
        acc_scr[...] += contrib

    @pl.when(f == nf - 1)
    def _():
        row_scr[...] = acc_scr[...] * w_ref[...]

        def scatter_body(r, _):
            tok = tok_ref[i * TILE + r]
            out_ref[tok, :] += row_scr[r, :]
            return 0
        jax.lax.fori_loop(0, TILE, scatter_body, 0, unroll=8)


def kernel(inputs, decoder_segment_ids, decoder_positions, gate_kernel, wi_0, wi_1, wo):
    del decoder_segment_ids, decoder_positions
    b, s, d = inputs.shape
    t = b * s
    n_exp = gate_kernel.shape[-1]
    f_dim = wi_0.shape[-1]
    k = 2
    nf = f_dim // FB
    nt = (t * k) // TILE + n_exp
    padrows = nt * TILE

    x = inputs.reshape(t, d)

    top_idx, top_w = pl.pallas_call(
        functools.partial(_router_kernel, n_exp=n_exp),
        out_shape=(
            jax.ShapeDtypeStruct((t, k), jnp.int32),
            jax.ShapeDtypeStruct((t, k), jnp.float32),
        ),
    )(x, gate_kernel)

    flat_e = top_idx.reshape(-1)
    flat_t = (jnp.arange(t * k, dtype=jnp.int32) // k)
    flat_w = top_w.reshape(-1)
    counts = jnp.bincount(flat_e, length=n_exp)
    padded = ((counts + TILE - 1) // TILE) * TILE
    pend = jnp.cumsum(padded)
    pstart = pend - padded
    ustart = jnp.cumsum(counts) - counts
    order = jnp.argsort(flat_e, stable=True)
    se = flat_e[order]
    pos = jnp.arange(t * k)
    dest = pstart[se] + (pos - ustart[se])
    sorted_tok = jnp.zeros(padrows, jnp.int32).at[dest].set(flat_t[order])
    sorted_w = jnp.zeros(padrows, jnp.float32).at[dest].set(flat_w[order])
    tile_grp = jnp.clip(
        jnp.searchsorted(pend, jnp.arange(nt) * TILE, side='right'),
        0, n_exp - 1).astype(jnp.int32)

    grid_spec = pltpu.PrefetchScalarGridSpec(
        num_scalar_prefetch=2,
        grid=(nt, nf),
        in_specs=[
            pl.BlockSpec((t, d), lambda i, f, grp, tok: (0, 0)),
            pl.BlockSpec((1, d, FB), lambda i, f, grp, tok: (grp[i], 0, f)),
            pl.BlockSpec((1, d, FB), lambda i, f, grp, tok: (grp[i], 0, f)),
            pl.BlockSpec((1, FB, d), lambda i, f, grp, tok: (grp[i], f, 0)),
            pl.BlockSpec((TILE, 1), lambda i, f, grp, tok: (i, 0)),
        ],
        out_specs=pl.BlockSpec((t, d), lambda i, f, grp, tok: (0, 0)),
        scratch_shapes=[
            pltpu.VMEM((TILE, d), jnp.float32),
            pltpu.VMEM((TILE, d), jnp.float32),
            pltpu.VMEM((TILE, d), jnp.float32),
        ],
    )

    out = pl.pallas_call(
        functools.partial(_gmm_kernel, nf=nf),
        grid_spec=grid_spec,
        out_shape=jax.ShapeDtypeStruct((t, d), jnp.float32),
        compiler_params=pltpu.CompilerParams(
            dimension_semantics=("arbitrary", "arbitrary"),
        ),
    )(tile_grp, sorted_tok, x,
      wi_0.astype(jnp.bfloat16), wi_1.astype(jnp.bfloat16),
      wo.astype(jnp.bfloat16), sorted_w.reshape(padrows, 1))

    return out.reshape(b, s, d)


# speedup vs baseline: 1.0301x; 1.0301x over previous
import functools

import jax
import jax.numpy as jnp
from jax import lax
from jax.experimental import pallas as pl
from jax.experimental.pallas import tpu as pltpu
from jax.experimental.pallas import tpu_sc as plsc

TILE = 256


def _router_kernel(x_ref, g_ref, idx_ref, w_ref, *, n_exp):
    logits = jnp.dot(x_ref[...], g_ref[...], preferred_element_type=jnp.float32)
    eidx = jax.lax.broadcasted_iota(jnp.int32, logits.shape, 1)
    m1 = jnp.max(logits, axis=1, keepdims=True)
    i1 = jnp.min(jnp.where(logits == m1, eidx, n_exp), axis=1, keepdims=True)
    masked = jnp.where(eidx == i1, -jnp.inf, logits)
    m2 = jnp.max(masked, axis=1, keepdims=True)
    i2 = jnp.min(jnp.where(masked == m2, eidx, n_exp), axis=1, keepdims=True)
    e2 = jnp.exp(m2 - m1)
    w1 = 1.0 / (1.0 + e2)
    w2 = e2 / (1.0 + e2)
    idx_ref[...] = jnp.concatenate([i1, i2], axis=1)
    w_ref[...] = jnp.concatenate([w1, w2], axis=1)


def _gmm_kernel(grp_ref, xs_ref, wi0_ref, wi1_ref, wo_ref, w_ref, outs_ref):
    xs = xs_ref[...].astype(jnp.bfloat16)
    a0 = jnp.dot(xs, wi0_ref[0], preferred_element_type=jnp.float32)
    a1 = jnp.dot(xs, wi1_ref[0], preferred_element_type=jnp.float32)
    h = (jax.nn.gelu(a0) * a1).astype(jnp.bfloat16)
    outs_ref[...] = jnp.dot(h, wo_ref[0],
                            preferred_element_type=jnp.float32) * w_ref[...]


def _sc_gather_rows(x, idx, padrows, d):
    info = plsc.get_sparse_core_info()
    nw = info.num_cores * info.num_subcores
    rows_per_w = padrows // nw
    chunk = 64
    n_chunks = rows_per_w // chunk
    mesh = plsc.VectorSubcoreMesh(core_axis_name="c", subcore_axis_name="s")

    @functools.partial(
        pl.kernel, mesh=mesh,
        out_type=jax.ShapeDtypeStruct((padrows, d), jnp.float32),
        scratch_types=[
            pltpu.VMEM((chunk,), jnp.int32),
            pltpu.VMEM((chunk, d), jnp.float32),
            pltpu.SemaphoreType.DMA,
        ],
    )
    def k(x_hbm, idx_hbm, xs_hbm, idx_v, rows_v, sem):
        wid = lax.axis_index("s") * info.num_cores + lax.axis_index("c")
        base = wid * rows_per_w

        def body(c, _):
            off = base + c * chunk
            pltpu.sync_copy(idx_hbm.at[pl.ds(off, chunk)], idx_v)
            pltpu.async_copy(x_hbm.at[idx_v], rows_v, sem).wait()
            pltpu.sync_copy(rows_v, xs_hbm.at[pl.ds(off, chunk)])
            return 0
        lax.fori_loop(0, n_chunks, body, 0)

    return k(x, idx)


def _sc_combine_rows(outs, pos0, pos1, t, d):
    info = plsc.get_sparse_core_info()
    nw = info.num_cores * info.num_subcores
    tok_per_w = t // nw
    chunk = 32
    n_chunks = tok_per_w // chunk
    lanes = info.num_lanes
    mesh = plsc.VectorSubcoreMesh(core_axis_name="c", subcore_axis_name="s")

    @functools.partial(
        pl.kernel, mesh=mesh,
        out_type=jax.ShapeDtypeStruct((t, d), jnp.float32),
        scratch_types=[
            pltpu.VMEM((chunk,), jnp.int32),
            pltpu.VMEM((chunk,), jnp.int32),
            pltpu.VMEM((chunk, d), jnp.float32),
            pltpu.VMEM((chunk, d), jnp.float32),
            pltpu.SemaphoreType.DMA,
            pltpu.SemaphoreType.DMA,
        ],
    )
    def k(outs_hbm, pos0_hbm, pos1_hbm, out_hbm,
          idx0_v, idx1_v, rows0_v, rows1_v, sem0, sem1):
        wid = lax.axis_index("s") * info.num_cores + lax.axis_index("c")
        base = wid * tok_per_w
        slices_per_row = d // lanes

        def body(c, _):
            off = base + c * chunk
            pltpu.sync_copy(pos0_hbm.at[pl.ds(off, chunk)], idx0_v)
            pltpu.sync_copy(pos1_hbm.at[pl.ds(off, chunk)], idx1_v)
            cp0 = pltpu.async_copy(outs_hbm.at[idx0_v], rows0_v, sem0)
            cp1 = pltpu.async_copy(outs_hbm.at[idx1_v], rows1_v, sem1)
            cp0.wait()
            cp1.wait()

            def add_row(r, _):
                for s in range(slices_per_row):
                    sl = pl.ds(s * lanes, lanes)
                    rows0_v[r, sl] = rows0_v[r, sl] + rows1_v[r, sl]
                return 0
            lax.fori_loop(0, chunk, add_row, 0)
            pltpu.sync_copy(rows0_v, out_hbm.at[pl.ds(off, chunk)])
            return 0
        lax.fori_loop(0, n_chunks, body, 0)

    return k(outs, pos0, pos1)


def kernel(inputs, decoder_segment_ids, decoder_positions, gate_kernel, wi_0, wi_1, wo):
    del decoder_segment_ids, decoder_positions
    b, s, d = inputs.shape
    t = b * s
    n_exp = gate_kernel.shape[-1]
    f_dim = wi_0.shape[-1]
    k = 2
    nt = (t * k) // TILE + n_exp
    padrows = nt * TILE

    x = inputs.reshape(t, d)

    top_idx, top_w = pl.pallas_call(
        functools.partial(_router_kernel, n_exp=n_exp),
        out_shape=(
            jax.ShapeDtypeStruct((t, k), jnp.int32),
            jax.ShapeDtypeStruct((t, k), jnp.float32),
        ),
    )(x, gate_kernel)

    flat_e = top_idx.reshape(-1)
    flat_t = (jnp.arange(t * k, dtype=jnp.int32) // k)
    flat_w = top_w.reshape(-1)
    counts = jnp.bincount(flat_e, length=n_exp)
    padded = ((counts + TILE - 1) // TILE) * TILE
    pend = jnp.cumsum(padded)
    pstart = pend - padded
    ustart = jnp.cumsum(counts) - counts
    order = jnp.argsort(flat_e, stable=True)
    se = flat_e[order]
    pos = jnp.arange(t * k)
    dest = pstart[se] + (pos - ustart[se])
    sorted_tok = jnp.zeros(padrows, jnp.int32).at[dest].set(flat_t[order])
    sorted_w = jnp.zeros(padrows, jnp.float32).at[dest].set(flat_w[order])
    rowpos = jnp.zeros(t * k, jnp.int32).at[order].set(
        dest.astype(jnp.int32)).reshape(t, k)
    tile_grp = jnp.clip(
        jnp.searchsorted(pend, jnp.arange(nt) * TILE, side='right'),
        0, n_exp - 1).astype(jnp.int32)

    xs = _sc_gather_rows(x, sorted_tok, padrows, d)

    grid_spec = pltpu.PrefetchScalarGridSpec(
        num_scalar_prefetch=1,
        grid=(nt,),
        in_specs=[
            pl.BlockSpec((TILE, d), lambda i, grp: (i, 0)),
            pl.BlockSpec((1, d, f_dim), lambda i, grp: (grp[i], 0, 0)),
            pl.BlockSpec((1, d, f_dim), lambda i, grp: (grp[i], 0, 0)),
            pl.BlockSpec((1, f_dim, d), lambda i, grp: (grp[i], 0, 0)),
            pl.BlockSpec((TILE, 1), lambda i, grp: (i, 0)),
        ],
        out_specs=pl.BlockSpec((TILE, d), lambda i, grp: (i, 0)),
    )
    outs = pl.pallas_call(
        _gmm_kernel,
        grid_spec=grid_spec,
        out_shape=jax.ShapeDtypeStruct((padrows, d), jnp.float32),
        compiler_params=pltpu.CompilerParams(
            dimension_semantics=("arbitrary",),
            vmem_limit_bytes=100 * 1024 * 1024,
        ),
    )(tile_grp, xs,
      wi_0.astype(jnp.bfloat16), wi_1.astype(jnp.bfloat16),
      wo.astype(jnp.bfloat16), sorted_w.reshape(padrows, 1))

    out = _sc_combine_rows(outs, rowpos[:, 0], rowpos[:, 1], t, d)
    return out.reshape(b, s, d)
